# baseline (speedup 1.0000x reference)
import functools

import jax
import jax.numpy as jnp
from jax import lax
from jax.experimental import pallas as pl
from jax.experimental.pallas import tpu as pltpu
from jax.experimental.pallas import tpu_sc as plsc

VOCAB = 100000
D = 128
NC, NS = 2, 16
NW = NC * NS
K = 128
NBUF = 4
SKEW = 2


def _emb_call(n, b_per_w, n_chunks):
    mesh = plsc.VectorSubcoreMesh(
        core_axis_name="c", subcore_axis_name="s",
        num_cores=NC, num_subcores=NS,
    )
    @functools.partial(
        pl.kernel,
        out_type=jax.ShapeDtypeStruct((n, D), jnp.float32),
        mesh=mesh,
        scratch_types=(
            [pltpu.VMEM((b_per_w,), jnp.int32),
             pltpu.VMEM((1, D), jnp.float32)]
            + [pltpu.VMEM((K, D), jnp.float32) for _ in range(NBUF)]
            + [pltpu.SemaphoreType.DMA for _ in range(2 * NBUF)]
        ),
    )
    def emb(tab_hbm, idx_hbm, zrow_hbm, out_hbm, idx_v, zbuf,
            *bufs_and_sems):
        bufs = bufs_and_sems[:NBUF]
        gsem = bufs_and_sems[NBUF:2 * NBUF]
        wsem = bufs_and_sems[2 * NBUF:]
        wid = lax.axis_index("s") * NC + lax.axis_index("c")
        base = wid * b_per_w
        pltpu.sync_copy(idx_hbm.at[pl.ds(base, b_per_w)], idx_v)
        pltpu.sync_copy(zrow_hbm, zbuf)

        def gissue(c, b):
            h = K // 4
            for p in range(4):
                pltpu.async_copy(
                    tab_hbm.at[idx_v.at[pl.ds(c * K + p * h, h)]],
                    bufs[b].at[pl.ds(p * h, h)], gsem[b])

        def gwait(b):
            pltpu.make_async_copy(
                tab_hbm.at[pl.ds(0, K)], bufs[b], gsem[b]).wait()

        def wissue(c, b):
            vecs = [idx_v[pl.ds(c * K + j * 16, 16)] for j in range(K // 16)]
            vmin = vecs[0]
            for v in vecs[1:]:
                vmin = jnp.minimum(vmin, v)
            min_tok = vmin[0]
            for l in range(1, 16):
                min_tok = jnp.minimum(min_tok, vmin[l])

            @pl.when(min_tok != 0)
            def _fast():
                pltpu.async_copy(
                    bufs[b], out_hbm.at[pl.ds(base + c * K, K)], wsem[b])

            @pl.when(min_tok == 0)
            def _padded():
                def grp_body(j, carry):
                    v = idx_v[pl.ds(c * K + j * 16, 16)]
                    for l in range(16):
                        r = j * 16 + l
                        dst = out_hbm.at[pl.ds(base + c * K + r, 1)]

                        @pl.when(v[l] == 0)
                        def _zero():
                            pltpu.async_copy(zbuf, dst, wsem[b])

                        @pl.when(v[l] != 0)
                        def _copy():
                            pltpu.async_copy(
                                bufs[b].at[pl.ds(r, 1)], dst, wsem[b])
                    return carry

                lax.fori_loop(0, K // 16, grp_body, 0)

        def wwait(b):
            pltpu.make_async_copy(
                bufs[b], out_hbm.at[pl.ds(base, K)], wsem[b]).wait()

        def stage(c, b, prime_c, need_wwait):
            gwait(b)
            wissue(c, b)
            if prime_c is not None:
                pb = (b + SKEW) % NBUF
                if need_wwait:
                    wwait(pb)
                gissue(prime_c, pb)

        for c in range(SKEW):
            gissue(c, c)

        for b in range(NBUF):
            stage(b, b, b + SKEW, b + SKEW >= NBUF)

        t_lim = (n_chunks - SKEW) // NBUF

        def body(t, carry):
            for b in range(NBUF):
                stage(t * NBUF + b, b, t * NBUF + b + SKEW, True)
            return carry

        lax.fori_loop(1, t_lim, body, 0)

        for c in range(t_lim * NBUF, n_chunks):
            pc = c + SKEW
            stage(c, c % NBUF, pc if pc < n_chunks else None, True)
        for b in range(NBUF):
            wwait(b)

    return emb


def kernel(tokens, table):
    bsz, seq = tokens.shape
    n = bsz * seq
    b_per_w = n // NW
    n_chunks = b_per_w // K
    assert n % NW == 0 and b_per_w % K == 0 and n_chunks >= NBUF + SKEW

    idx = tokens.reshape(-1).astype(jnp.int32)
    zrow = jnp.zeros((1, D), jnp.float32)
    out = _emb_call(n, b_per_w, n_chunks)(table, idx, zrow)
    return out.reshape(bsz, seq, D)


# speedup vs baseline: 1.0043x; 1.0043x over previous
import functools

import jax
import jax.numpy as jnp
from jax import lax
from jax.experimental import pallas as pl
from jax.experimental.pallas import tpu as pltpu
from jax.experimental.pallas import tpu_sc as plsc

VOCAB = 100000
D = 128
NC, NS = 2, 16
NW = NC * NS
K = 128
NBUF = 4
SKEW = 2


def _emb_call(n, b_per_w, n_chunks):
    mesh = plsc.VectorSubcoreMesh(
        core_axis_name="c", subcore_axis_name="s",
        num_cores=NC, num_subcores=NS,
    )
    @functools.partial(
        pl.kernel,
        out_type=jax.ShapeDtypeStruct((n, D), jnp.float32),
        mesh=mesh,
        scratch_types=(
            [pltpu.VMEM((b_per_w,), jnp.int32),
             pltpu.VMEM((1, D), jnp.float32)]
            + [pltpu.VMEM((K, D), jnp.float32) for _ in range(NBUF)]
            + [pltpu.SemaphoreType.DMA for _ in range(2 * NBUF)]
        ),
    )
    def emb(tab_hbm, idx_hbm, zrow_hbm, out_hbm, idx_v, zbuf,
            *bufs_and_sems):
        bufs = bufs_and_sems[:NBUF]
        gsem = bufs_and_sems[NBUF:2 * NBUF]
        wsem = bufs_and_sems[2 * NBUF:]
        wid = lax.axis_index("s") * NC + lax.axis_index("c")
        base = wid * b_per_w
        pltpu.sync_copy(idx_hbm.at[pl.ds(base, b_per_w)], idx_v)
        pltpu.sync_copy(zrow_hbm, zbuf)

        def gissue(c, b):
            h = K // 2
            pltpu.async_copy(
                tab_hbm.at[idx_v.at[pl.ds(c * K, h)]],
                bufs[b].at[pl.ds(0, h)], gsem[b])
            pltpu.async_copy(
                tab_hbm.at[idx_v.at[pl.ds(c * K + h, h)]],
                bufs[b].at[pl.ds(h, h)], gsem[b])

        def gwait(b):
            pltpu.make_async_copy(
                tab_hbm.at[pl.ds(0, K)], bufs[b], gsem[b]).wait()

        def wissue(c, b):
            vecs = [idx_v[pl.ds(c * K + j * 16, 16)] for j in range(K // 16)]
            vmin = vecs[0]
            for v in vecs[1:]:
                vmin = jnp.minimum(vmin, v)
            min_tok = vmin[0]
            for l in range(1, 16):
                min_tok = jnp.minimum(min_tok, vmin[l])

            @pl.when(min_tok != 0)
            def _fast():
                pltpu.async_copy(
                    bufs[b], out_hbm.at[pl.ds(base + c * K, K)], wsem[b])

            @pl.when(min_tok == 0)
            def _padded():
                def grp_body(j, carry):
                    v = idx_v[pl.ds(c * K + j * 16, 16)]
                    for l in range(16):
                        r = j * 16 + l
                        dst = out_hbm.at[pl.ds(base + c * K + r, 1)]

                        @pl.when(v[l] == 0)
                        def _zero():
                            pltpu.async_copy(zbuf, dst, wsem[b])

                        @pl.when(v[l] != 0)
                        def _copy():
                            pltpu.async_copy(
                                bufs[b].at[pl.ds(r, 1)], dst, wsem[b])
                    return carry

                lax.fori_loop(0, K // 16, grp_body, 0)

        def wwait(b):
            pltpu.make_async_copy(
                bufs[b], out_hbm.at[pl.ds(base, K)], wsem[b]).wait()

        def stage(c, b, prime_c, need_wwait):
            gwait(b)
            wissue(c, b)
            if prime_c is not None:
                pb = (b + SKEW) % NBUF
                if need_wwait:
                    wwait(pb)
                gissue(prime_c, pb)

        for c in range(SKEW):
            gissue(c, c)

        for b in range(NBUF):
            stage(b, b, b + SKEW, b + SKEW >= NBUF)

        t_lim = (n_chunks - SKEW) // NBUF

        def body(t, carry):
            for b in range(NBUF):
                stage(t * NBUF + b, b, t * NBUF + b + SKEW, True)
            return carry

        lax.fori_loop(1, t_lim, body, 0)

        for c in range(t_lim * NBUF, n_chunks):
            pc = c + SKEW
            stage(c, c % NBUF, pc if pc < n_chunks else None, True)
        for b in range(NBUF):
            wwait(b)

    return emb


def kernel(tokens, table):
    bsz, seq = tokens.shape
    n = bsz * seq
    b_per_w = n // NW
    n_chunks = b_per_w // K
    assert n % NW == 0 and b_per_w % K == 0 and n_chunks >= NBUF + SKEW

    idx = tokens.reshape(-1).astype(jnp.int32)
    zrow = jnp.zeros((1, D), jnp.float32)
    out = _emb_call(n, b_per_w, n_chunks)(table, idx, zrow)
    return out.reshape(bsz, seq, D)
